# Initial kernel scaffold; baseline (speedup 1.0000x reference)
import jax
import jax.numpy as jnp
from jax.experimental import pallas as pl


def kernel(X, W_sp, b_sp, W_sd, b_sd, W_mp, b_mp, W_md, b_md, W_map, b_map, user_ratings, user_personalities, top_map, mid_map):
    raise NotImplementedError("write your pallas kernel here")


# speedup vs baseline: 3.2534x; 3.2534x over previous
import functools

import jax
import jax.numpy as jnp
from jax import lax
from jax.experimental import pallas as pl
from jax.experimental.pallas import tpu as pltpu
from jax.experimental.pallas import tpu_sc as plsc

B = 32
D_P = 50
D_LAT = 128
N_SMALL = 500
N_MID = 2000
N_USERS = 2000
N_ITEMS = 10000
K20 = 20
K40 = 40
NN_K = 50
NJ = 64
NCH = 5
CHW = 2048
N_PAD = NCH * CHW

NEG_INF = float("-inf")
BIG_I = 2 ** 30


def _select_topk(work, keys, k, out_dtype=jnp.float32):
    n_b = work.shape[0]
    iota_k = lax.broadcasted_iota(jnp.int32, (n_b, k), 1)

    def body(t, carry):
        wk, sv, sk = carry
        m = jnp.max(wk, axis=1, keepdims=True)
        eq = wk == m
        kmin = jnp.min(jnp.where(eq, keys, BIG_I), axis=1, keepdims=True)
        hit = iota_k == t
        sv = jnp.where(hit, m.astype(out_dtype), sv)
        sk = jnp.where(hit, kmin, sk)
        wk = jnp.where(eq & (keys == kmin), NEG_INF, wk)
        return wk, sv, sk

    sv0 = jnp.zeros((n_b, k), out_dtype)
    sk0 = jnp.zeros((n_b, k), jnp.int32)
    _, sv, sk = lax.fori_loop(0, k, body, (work, sv0, sk0))
    return sv, sk


def _winner_valid(map_row, map_col, n, blk):
    key_row = map_row * 16384 + lax.broadcasted_iota(jnp.int32, (1, n), 1)
    gmax = jnp.full((1, n), -1, jnp.int32)
    for s in range(0, n, blk):
        mc = lax.slice(map_col, (s, 0), (s + blk, 1))
        kc = mc * 16384 + (lax.broadcasted_iota(jnp.int32, (blk, 1), 0) + s)
        eq = mc == map_row
        part = jnp.max(jnp.where(eq, kc, -1), axis=0, keepdims=True)
        gmax = jnp.maximum(gmax, part)
    return gmax == key_row


def _body_pre(x_ref, wsp_ref, bsp_ref, wsd_ref, bsd_ref, wmp_ref, bmp_ref,
              wmd_ref, bmd_ref, wmap_ref, bmap_ref, pers_ref,
              tmap_r_ref, tmap_c_ref, mmap_r_ref, mmap_c_ref,
              tv_ref, ti_ref, vb_ref, den_ref, probs_ref,
              tvals_ref, titems_ref, mvals_ref, mitems_ref):
    x = x_ref[...]
    xn = x / (jnp.sqrt(jnp.sum(x * x, axis=1, keepdims=True)) + 1e-8)
    p = pers_ref[0]
    pn = p / (jnp.sqrt(jnp.sum(p * p, axis=1, keepdims=True)) + 1e-8)
    sims = lax.dot_general(xn, pn, (((1,), (1,)), ((), ())),
                           preferred_element_type=jnp.float32)
    lane = lax.broadcasted_iota(jnp.int32, (1, N_USERS), 1)
    sv, si = _select_topk(sims, lane, NN_K)
    pad = jnp.zeros((B, NJ - NN_K), jnp.float32)
    tv64 = jnp.concatenate([sv, pad], axis=1)
    ti64 = jnp.concatenate([si, pad.astype(jnp.int32)], axis=1)
    tv_ref[...] = tv64
    ti_ref[...] = ti64
    vb_ref[...] = jnp.broadcast_to(tv64[:, :, None], (B, NJ, 16))
    den_ref[...] = jnp.broadcast_to(
        jnp.sum(sv, axis=1, keepdims=True) + 1e-8, (B, 128))

    z_s = jnp.tanh(jnp.dot(x, wsp_ref[...],
                           preferred_element_type=jnp.float32) + bsp_ref[...])
    ps = jnp.dot(z_s, wsd_ref[...],
                 preferred_element_type=jnp.float32) + bsd_ref[...]
    z_m = jnp.tanh(jnp.dot(x, wmp_ref[...],
                           preferred_element_type=jnp.float32) + bmp_ref[...])
    pm = jnp.dot(z_m, wmd_ref[...],
                 preferred_element_type=jnp.float32) + bmd_ref[...]

    lg = jnp.dot(x, wmap_ref[...],
                 preferred_element_type=jnp.float32) + bmap_ref[...]
    lg = lg - jnp.max(lg, axis=1, keepdims=True)
    el = jnp.exp(lg)
    probs_ref[...] = el / jnp.sum(el, axis=1, keepdims=True)

    tv = _winner_valid(tmap_r_ref[...], tmap_c_ref[...], N_SMALL, 500)
    mv = _winner_valid(mmap_r_ref[...], mmap_c_ref[...], N_MID, 400)
    tvals, titems = _select_topk(
        jnp.where(tv, ps, NEG_INF), tmap_r_ref[...], K40)
    mvals, mitems = _select_topk(
        jnp.where(mv, pm, NEG_INF), mmap_r_ref[...], K40)
    tvals_ref[...] = tvals
    titems_ref[...] = titems
    mvals_ref[...] = mvals
    mitems_ref[...] = mitems


def _knn_sc(rat_hbm, ti_hbm, vb_hbm, out_hbm,
            idx_v, idxc_v, vb_v, g0, g1, s_v, o_v, sem0, sem1):
    cid = lax.axis_index("c")
    sid = lax.axis_index("s")
    b = sid * 2 + cid
    pltpu.sync_copy(ti_hbm.at[b], idx_v)
    pltpu.sync_copy(vb_hbm.at[b], vb_v)
    for c in range(NCH):
        for t in range(NJ // 16):
            sl = pl.ds(16 * t, 16)
            idxc_v[sl] = idx_v[sl] * NCH + c
        cp = pltpu.async_copy(rat_hbm.at[idxc_v.at[pl.ds(0, 8)]], g0, sem0)
        for k in range(8):
            cur, nxt = (g0, g1) if k % 2 == 0 else (g1, g0)
            nsem = sem1 if k % 2 == 0 else sem0
            ncp = None
            if k < 7:
                ncp = pltpu.async_copy(
                    rat_hbm.at[idxc_v.at[pl.ds(8 * (k + 1), 8)]], nxt, nsem)
            cp.wait()

            def acc_t(t, _, k=k, cur=cur):
                sl = pl.ds(16 * t, 16)
                for s in range(8):
                    prod = vb_v[8 * k + s] * cur[s, sl]
                    if k == 0:
                        s_v[s, sl] = prod
                    else:
                        s_v[s, sl] = s_v[s, sl] + prod
                return 0

            lax.fori_loop(0, CHW // 16, acc_t, 0)
            cp = ncp

        def tree_t(t, _):
            sl = pl.ds(16 * t, 16)
            a = s_v[0, sl] + s_v[4, sl]
            bb = s_v[2, sl] + s_v[6, sl]
            cc = s_v[1, sl] + s_v[5, sl]
            dd = s_v[3, sl] + s_v[7, sl]
            o_v[sl] = (a + bb) + (cc + dd)
            return 0

        lax.fori_loop(0, CHW // 16, tree_t, 0)
        pltpu.sync_copy(o_v, out_hbm.at[b * NCH + c])


def _body_post(ksum_ref, den_ref, probs_ref, tvals_ref, titems_ref,
               mvals_ref, mitems_ref, out_ref):
    kp = ksum_ref[...] / den_ref[:, 0:1]
    lane = lax.broadcasted_iota(jnp.int32, (1, N_PAD), 1)
    kp = jnp.where(lane < N_ITEMS, kp, NEG_INF)
    kvals, kitems = _select_topk(kp, lane, K40)
    probs = probs_ref[...]
    cvals = jnp.concatenate(
        [tvals_ref[...] * probs[:, 0:1], mvals_ref[...] * probs[:, 1:2],
         kvals * probs[:, 2:3]], axis=1)
    citems = jnp.concatenate([titems_ref[...], mitems_ref[...], kitems],
                             axis=1)
    eq3 = citems[:, :, None] == citems[:, None, :]
    fused = jnp.sum(jnp.where(eq3, cvals[:, None, :], 0.0), axis=2)
    j_lt_i = (lax.broadcasted_iota(jnp.int32, (1, 120, 120), 2)
              < lax.broadcasted_iota(jnp.int32, (1, 120, 120), 1))
    first = ~jnp.any(eq3 & j_lt_i, axis=2)
    fvals, fitems = _select_topk(
        jnp.where(first, fused, NEG_INF), citems, K20)
    out_ref[...] = fitems.astype(jnp.float32)


def kernel(X, W_sp, b_sp, W_sd, b_sd, W_mp, b_mp, W_md, b_md, W_map, b_map,
           user_ratings, user_personalities, top_map, mid_map):
    tmap = top_map.astype(jnp.int32)
    mmap = mid_map.astype(jnp.int32)

    pre = pl.pallas_call(
        _body_pre,
        out_shape=[
            jax.ShapeDtypeStruct((B, NJ), jnp.float32),
            jax.ShapeDtypeStruct((B, NJ), jnp.int32),
            jax.ShapeDtypeStruct((B, NJ, 16), jnp.float32),
            jax.ShapeDtypeStruct((B, 128), jnp.float32),
            jax.ShapeDtypeStruct((B, 3), jnp.float32),
            jax.ShapeDtypeStruct((B, K40), jnp.float32),
            jax.ShapeDtypeStruct((B, K40), jnp.int32),
            jax.ShapeDtypeStruct((B, K40), jnp.float32),
            jax.ShapeDtypeStruct((B, K40), jnp.int32),
        ],
    )
    (tv64, ti64, vb, den, probs, tvals, titems, mvals, mitems) = pre(
        X, W_sp, b_sp.reshape(1, D_LAT), W_sd, b_sd.reshape(1, N_SMALL),
        W_mp, b_mp.reshape(1, D_LAT), W_md, b_md.reshape(1, N_MID),
        W_map, b_map.reshape(1, 3), user_personalities,
        tmap.reshape(1, N_SMALL), tmap.reshape(N_SMALL, 1),
        mmap.reshape(1, N_MID), mmap.reshape(N_MID, 1))

    ratp = jnp.pad(user_ratings.reshape(N_USERS, N_ITEMS),
                   ((0, 0), (0, N_PAD - N_ITEMS)))
    rat2 = ratp.reshape(N_USERS * NCH, CHW)
    knn = pl.kernel(
        _knn_sc,
        out_type=jax.ShapeDtypeStruct((B * NCH, CHW), jnp.float32),
        mesh=plsc.VectorSubcoreMesh(core_axis_name="c", subcore_axis_name="s"),
        scratch_types=[
            pltpu.VMEM((NJ,), jnp.int32),
            pltpu.VMEM((NJ,), jnp.int32),
            pltpu.VMEM((NJ, 16), jnp.float32),
            pltpu.VMEM((8, CHW), jnp.float32),
            pltpu.VMEM((8, CHW), jnp.float32),
            pltpu.VMEM((8, CHW), jnp.float32),
            pltpu.VMEM((CHW,), jnp.float32),
            pltpu.SemaphoreType.DMA,
            pltpu.SemaphoreType.DMA,
        ],
    )
    ksum = knn(rat2, ti64, vb).reshape(B, N_PAD)

    post = pl.pallas_call(
        _body_post,
        out_shape=jax.ShapeDtypeStruct((B, K20), jnp.float32),
    )
    return post(ksum, den, probs, tvals, titems, mvals, mitems)
